# baseline (speedup 1.0000x reference)
import functools

import jax
import jax.numpy as jnp
import numpy as np
from jax import lax
from jax.experimental import pallas as pl
from jax.experimental.pallas import tpu as pltpu
from jax.experimental.pallas import tpu_sc as plsc

NN = 10000
EE = 320000
N_PAD = 10240
ROWS_PT = N_PAD // 16
BPT = 160
NBT = 16 * BPT
E_PAD = NBT * 128
FW = 64
RB = 512
GRID_R = N_PAD // RB
BN_EPS = 1e-5


def _sc_mesh():
    return plsc.VectorSubcoreMesh(core_axis_name="c", subcore_axis_name="s")


def _fill_const(buf, rows, cols, val):
    def frow(r, carry):
        def fcol(k, c2):
            buf[r, pl.ds(k * 16, 16)] = jnp.full((16,), val, jnp.float32)
            return c2
        return lax.fori_loop(0, cols // 16, fcol, carry)
    lax.fori_loop(0, rows, frow, 0)


def _deg_call(dst_all):
    scratch = [
        pltpu.VMEM((128, 16), jnp.float32),
        pltpu.VMEM((128, 16), jnp.float32),
        pltpu.VMEM((BPT, 128), jnp.int32),
        pltpu.MemorySpace.VMEM_SHARED((N_PAD, 16), jnp.float32),
    ]

    @functools.partial(
        pl.kernel,
        out_type=jax.ShapeDtypeStruct((2 * N_PAD, 16), jnp.float32),
        mesh=_sc_mesh(),
        scratch_types=scratch,
        compiler_params=pltpu.CompilerParams(use_tc_tiling_on_sc=False),
    )
    def deg_kernel(dst_hbm, deg_hbm, ones_v, zeros_v, idx_v, acc):
        c = lax.axis_index("c")
        s = lax.axis_index("s")
        _fill_const(ones_v, 128, 16, 1.0)
        _fill_const(zeros_v, 128, 16, 0.0)

        def zslab(j, carry):
            pltpu.sync_copy(zeros_v, acc.at[pl.ds(s * ROWS_PT + j * 128, 128)])
            return carry
        lax.fori_loop(0, ROWS_PT // 128, zslab, 0)
        plsc.subcore_barrier()

        pltpu.sync_copy(dst_hbm.at[pl.ds(c * NBT + s * BPT, BPT)], idx_v)

        def scat(i, carry):
            pltpu.sync_copy(ones_v, acc.at[idx_v.at[i]], add=True)
            return carry
        lax.fori_loop(0, BPT, scat, 0)
        plsc.subcore_barrier()

        pltpu.sync_copy(
            acc.at[pl.ds(s * ROWS_PT, ROWS_PT)],
            deg_hbm.at[pl.ds(c * N_PAD + s * ROWS_PT, ROWS_PT)])

    return deg_kernel(dst_all)


def _make_agg(nch):
    nvi = 2 * nch
    NB = 5
    scratch = (
        [
            pltpu.VMEM((BPT, 128), jnp.int32),
            pltpu.VMEM((BPT, 128), jnp.int32),
        ]
        + [pltpu.VMEM((128, FW), jnp.float32) for _ in range(NB)]
        + [pltpu.MemorySpace.VMEM_SHARED((N_PAD, FW), jnp.float32)]
        + [pltpu.SemaphoreType.DMA for _ in range(2 * NB)]
    )

    @functools.partial(
        pl.kernel,
        out_type=jax.ShapeDtypeStruct((nvi, N_PAD, FW), jnp.float32),
        mesh=_sc_mesh(),
        scratch_types=scratch,
        compiler_params=pltpu.CompilerParams(use_tc_tiling_on_sc=False),
    )
    def agg_kernel(u_hbm, src_hbm, dst_hbm, out_hbm, *refs):
        srcb, dstb = refs[0], refs[1]
        rows = refs[2:2 + NB]
        acc = refs[2 + NB]
        gsem = refs[3 + NB:3 + 2 * NB]
        ssem = refs[3 + 2 * NB:3 + 3 * NB]
        c = lax.axis_index("c")
        s = lax.axis_index("s")
        base_b = c * NBT + s * BPT
        pltpu.sync_copy(src_hbm.at[pl.ds(base_b, BPT)], srcb)
        pltpu.sync_copy(dst_hbm.at[pl.ds(base_b, BPT)], dstb)

        def do_chunk(vi):
            uview = u_hbm.at[vi]

            _fill_const(rows[NB - 1], 128, FW, 0.0)

            def zslab(j, carry):
                pltpu.sync_copy(rows[NB - 1],
                                acc.at[pl.ds(s * ROWS_PT + j * 128, 128)])
                return carry
            lax.fori_loop(0, ROWS_PT // 128, zslab, 0)
            plsc.subcore_barrier()

            for t in range(4):
                pltpu.async_copy(uview.at[srcb.at[t]], rows[t], gsem[t])

            def grp(j, carry):
                b0 = NB * j
                for u in range(NB):
                    b = b0 + u
                    tn = (u + 4) % NB

                    pltpu.make_async_copy(
                        uview.at[srcb.at[b]], rows[u], gsem[u]).wait()
                    pltpu.async_copy(rows[u], acc.at[dstb.at[b]], ssem[u],
                                     add=True)

                    @pl.when(b < BPT - 4)
                    def _prefetch():
                        @pl.when(b >= 1)
                        def _drain_prev():
                            pltpu.make_async_copy(
                                rows[tn], acc.at[dstb.at[b - 1]],
                                ssem[tn]).wait()
                        pltpu.async_copy(
                            uview.at[srcb.at[b + 4]], rows[tn], gsem[tn])
                return carry
            lax.fori_loop(0, BPT // NB, grp, 0)
            for t in range(NB):
                pltpu.make_async_copy(
                    rows[t], acc.at[dstb.at[BPT - NB + t]], ssem[t]).wait()
            plsc.subcore_barrier()

            pltpu.sync_copy(acc.at[pl.ds(s * ROWS_PT, ROWS_PT)],
                            out_hbm.at[vi].at[pl.ds(s * ROWS_PT, ROWS_PT)])
            plsc.subcore_barrier()

        @pl.when(c == 0)
        def _core0():
            for ch in range(nch):
                do_chunk(ch)

        @pl.when(c == 1)
        def _core1():
            for ch in range(nch):
                do_chunk(nch + ch)

    return agg_kernel


def _embed_call(x_pad, w, b2d, deg1, deg2):
    def body(x_ref, w_ref, b_ref, d1_ref, d2_ref,
             h_ref, v_ref, i1_ref, i2_ref):
        i = pl.program_id(0)
        h = jnp.dot(x_ref[...], w_ref[...], preferred_element_type=jnp.float32)
        h = jnp.maximum(h + b_ref[...], 0.0)
        row = i * RB + lax.broadcasted_iota(jnp.int32, (RB, 16), 0)
        valid = row < NN
        d1 = d1_ref[...]
        d2 = d2_ref[...]
        dinv1 = jnp.where((d1 > 0) & valid, lax.rsqrt(d1), 0.0)
        dinv2 = jnp.where((d2 > 0) & valid, lax.rsqrt(d2), 0.0)
        h_ref[...] = h
        i1_ref[...] = dinv1
        i2_ref[...] = dinv2
        c1 = dinv1[:, :1]
        c2 = dinv2[:, :1]
        v_ref[0, 0] = h[:, :FW] * c1
        v_ref[0, 1] = h[:, FW:] * c1
        v_ref[1, 0] = h[:, :FW] * c2
        v_ref[1, 1] = h[:, FW:] * c2

    return pl.pallas_call(
        body,
        grid=(GRID_R,),
        in_specs=[
            pl.BlockSpec((RB, 128), lambda i: (i, 0)),
            pl.BlockSpec((128, 128), lambda i: (0, 0)),
            pl.BlockSpec((1, 128), lambda i: (0, 0)),
            pl.BlockSpec((RB, 16), lambda i: (i, 0)),
            pl.BlockSpec((RB, 16), lambda i: (i, 0)),
        ],
        out_specs=[
            pl.BlockSpec((RB, 128), lambda i: (i, 0)),
            pl.BlockSpec((2, 2, RB, FW), lambda i: (0, 0, i, 0)),
            pl.BlockSpec((RB, 16), lambda i: (i, 0)),
            pl.BlockSpec((RB, 16), lambda i: (i, 0)),
        ],
        out_shape=[
            jax.ShapeDtypeStruct((N_PAD, 128), jnp.float32),
            jax.ShapeDtypeStruct((2, 2, N_PAD, FW), jnp.float32),
            jax.ShapeDtypeStruct((N_PAD, 16), jnp.float32),
            jax.ShapeDtypeStruct((N_PAD, 16), jnp.float32),
        ],
    )(x_pad, w, b2d, deg1, deg2)


def _scale_bn_call(nch, acc, dinv1, dinv2, gamma3d, beta3d):
    nvi = nch
    bn_scale = float(1.0 / np.sqrt(1.0 + BN_EPS))

    def body(a_ref, d1_ref, d2_ref, g_ref, b_ref, z_ref, v_ref):
        ch = pl.program_id(0)
        a = jnp.concatenate([a_ref[0], a_ref[1]], axis=1)
        d1 = d1_ref[...][:, :1]
        d2 = d2_ref[...][:, :1]
        dsel = jnp.where(2 * ch < nvi, d1, d2)
        z = a * dsel * (g_ref[0, 0] * bn_scale) + b_ref[0, 0]
        z_ref[0] = z
        v_ref[0, 0] = z[:, :FW] * d1
        v_ref[0, 1] = z[:, FW:] * d1
        v_ref[1, 0] = z[:, :FW] * d2
        v_ref[1, 1] = z[:, FW:] * d2

    return pl.pallas_call(
        body,
        grid=(nvi, GRID_R),
        in_specs=[
            pl.BlockSpec((2, RB, FW), lambda ch, r: (ch, r, 0)),
            pl.BlockSpec((RB, 16), lambda ch, r: (r, 0)),
            pl.BlockSpec((RB, 16), lambda ch, r: (r, 0)),
            pl.BlockSpec((1, 1, 128), lambda ch, r: (ch, 0, 0)),
            pl.BlockSpec((1, 1, 128), lambda ch, r: (ch, 0, 0)),
        ],
        out_specs=[
            pl.BlockSpec((1, RB, 128), lambda ch, r: (ch, r, 0)),
            pl.BlockSpec((2, 2, RB, FW), lambda ch, r: (0, ch, r, 0)),
        ],
        out_shape=[
            jax.ShapeDtypeStruct((nvi, N_PAD, 128), jnp.float32),
            jax.ShapeDtypeStruct((2, 2 * nvi, N_PAD, FW), jnp.float32),
        ],
    )(acc, dinv1, dinv2, gamma3d, beta3d)


def _bn2_project(acc, dinv1, dinv2, g64, b64, wa):
    bn_scale = float(1.0 / np.sqrt(1.0 + BN_EPS))

    def body(a_ref, d1_ref, d2_ref, g_ref, b_ref, wa_ref, z_ref, v_ref):
        d1 = d1_ref[...][:, :1]
        d2 = d2_ref[...][:, :1]
        p1 = jnp.zeros((RB, 128), jnp.float32)
        p2 = jnp.zeros((RB, 128), jnp.float32)
        zs = []
        for k in range(8):
            dsel = d1 if k < 4 else d2
            z64 = a_ref[k] * dsel * (g_ref[k] * bn_scale) + b_ref[k]
            zs.append(z64)
            p1 += jnp.dot(z64, wa_ref[k], preferred_element_type=jnp.float32)
            p2 += jnp.dot(z64, wa_ref[8 + k],
                          preferred_element_type=jnp.float32)
        for ch in range(4):
            z_ref[ch] = jnp.concatenate([zs[2 * ch], zs[2 * ch + 1]], axis=1)
        v_ref[0, 0] = p1[:, :FW] * d1
        v_ref[0, 1] = p1[:, FW:] * d1
        v_ref[1, 0] = p2[:, :FW] * d2
        v_ref[1, 1] = p2[:, FW:] * d2

    return pl.pallas_call(
        body,
        grid=(GRID_R,),
        in_specs=[
            pl.BlockSpec((8, RB, FW), lambda i: (0, i, 0)),
            pl.BlockSpec((RB, 16), lambda i: (i, 0)),
            pl.BlockSpec((RB, 16), lambda i: (i, 0)),
            pl.BlockSpec((8, 1, FW), lambda i: (0, 0, 0)),
            pl.BlockSpec((8, 1, FW), lambda i: (0, 0, 0)),
            pl.BlockSpec((16, FW, 128), lambda i: (0, 0, 0)),
        ],
        out_specs=[
            pl.BlockSpec((4, RB, 128), lambda i: (0, i, 0)),
            pl.BlockSpec((2, 2, RB, FW), lambda i: (0, 0, i, 0)),
        ],
        out_shape=[
            jax.ShapeDtypeStruct((4, N_PAD, 128), jnp.float32),
            jax.ShapeDtypeStruct((2, 2, N_PAD, FW), jnp.float32),
        ],
    )(acc, dinv1, dinv2, g64, b64, wa)


def _final_call(h, z1, z2, a3, dinv1, dinv2, w7, b2d):
    def body(h_ref, z1_ref, z2_ref, a3_ref, d1_ref, d2_ref, w7_ref,
             b_ref, o_ref):
        s = jnp.dot(h_ref[...], w7_ref[0], preferred_element_type=jnp.float32)
        for k in range(2):
            s += jnp.dot(z1_ref[k], w7_ref[1 + k],
                         preferred_element_type=jnp.float32)
        for k in range(4):
            s += jnp.dot(z2_ref[k], w7_ref[3 + k],
                         preferred_element_type=jnp.float32)
        d1 = d1_ref[...][:, :1]
        d2 = d2_ref[...][:, :1]
        q1 = jnp.concatenate([a3_ref[0], a3_ref[1]], axis=1)
        q2 = jnp.concatenate([a3_ref[2], a3_ref[3]], axis=1)
        o_ref[...] = s + q1 * d1 + q2 * d2 + b_ref[...]

    return pl.pallas_call(
        body,
        grid=(GRID_R,),
        in_specs=[
            pl.BlockSpec((RB, 128), lambda i: (i, 0)),
            pl.BlockSpec((2, RB, 128), lambda i: (0, i, 0)),
            pl.BlockSpec((4, RB, 128), lambda i: (0, i, 0)),
            pl.BlockSpec((4, RB, FW), lambda i: (0, i, 0)),
            pl.BlockSpec((RB, 16), lambda i: (i, 0)),
            pl.BlockSpec((RB, 16), lambda i: (i, 0)),
            pl.BlockSpec((7, 128, 128), lambda i: (0, 0, 0)),
            pl.BlockSpec((1, 128), lambda i: (0, 0)),
        ],
        out_specs=pl.BlockSpec((RB, 128), lambda i: (i, 0)),
        out_shape=jax.ShapeDtypeStruct((N_PAD, 128), jnp.float32),
    )(h, z1, z2, a3, dinv1, dinv2, w7, b2d)


def _prep_edges(edge_index):
    pad = E_PAD - EE
    src = jnp.concatenate(
        [edge_index[0], jnp.full((pad,), NN, jnp.int32)]).reshape(NBT, 128)
    dst = jnp.concatenate(
        [edge_index[1], jnp.full((pad,), NN, jnp.int32)]).reshape(NBT, 128)
    return src, dst


def kernel(x, edge_index, edge_index2, W_embed, b_embed,
           bn0_gamma, bn0_beta, bn1_gamma, bn1_beta, W_out, b_out):
    s1, d1 = _prep_edges(edge_index)
    s2, d2 = _prep_edges(edge_index2)
    src_all = jnp.concatenate([s1, s2], axis=0)
    dst_all = jnp.concatenate([d1, d2], axis=0)

    deg_flat = _deg_call(dst_all)
    deg1 = deg_flat[:N_PAD]
    deg2 = deg_flat[N_PAD:]

    x_pad = jnp.pad(x, ((0, N_PAD - NN), (0, 0)))
    h, v0, dinv1, dinv2 = _embed_call(
        x_pad, W_embed, b_embed.reshape(1, 128), deg1, deg2)

    agg2 = _make_agg(2)
    acc1 = agg2(v0.reshape(4, N_PAD, FW), src_all, dst_all)
    z1, v1 = _scale_bn_call(2, acc1, dinv1, dinv2,
                            bn0_gamma.reshape(2, 1, 128),
                            bn0_beta.reshape(2, 1, 128))

    acc2 = _make_agg(4)(v1.reshape(8, N_PAD, FW), src_all, dst_all)
    z2, v3 = _bn2_project(acc2, dinv1, dinv2,
                          bn1_gamma.reshape(8, 1, FW),
                          bn1_beta.reshape(8, 1, FW),
                          W_out[896:].reshape(16, FW, 128))

    acc3 = agg2(v3.reshape(4, N_PAD, FW), src_all, dst_all)

    out = _final_call(h, z1, z2, acc3, dinv1, dinv2,
                      W_out[:896].reshape(7, 128, 128),
                      b_out.reshape(1, 128))
    return out[:NN]


# speedup vs baseline: 1.6780x; 1.6780x over previous
import functools

import jax
import jax.numpy as jnp
import numpy as np
from jax import lax
from jax.experimental import pallas as pl
from jax.experimental.pallas import tpu as pltpu
from jax.experimental.pallas import tpu_sc as plsc

NN = 10000
EE = 320000
N_PAD = 10240
ROWS_PT = N_PAD // 16
BPT = 160
NBT = 16 * BPT
E_PAD = NBT * 128
FW = 32
RB = 512
GRID_R = N_PAD // RB
BN_EPS = 1e-5


def _sc_mesh():
    return plsc.VectorSubcoreMesh(core_axis_name="c", subcore_axis_name="s")


def _fill_const(buf, rows, cols, val):
    def frow(r, carry):
        def fcol(k, c2):
            buf[r, pl.ds(k * 16, 16)] = jnp.full((16,), val, jnp.float32)
            return c2
        return lax.fori_loop(0, cols // 16, fcol, carry)
    lax.fori_loop(0, rows, frow, 0)


def _deg_call(dst_all):
    scratch = [
        pltpu.VMEM((128, 16), jnp.float32),
        pltpu.VMEM((128, 16), jnp.float32),
        pltpu.VMEM((BPT, 128), jnp.int32),
        pltpu.MemorySpace.VMEM_SHARED((N_PAD, 16), jnp.float32),
    ]

    @functools.partial(
        pl.kernel,
        out_type=jax.ShapeDtypeStruct((2 * N_PAD, 16), jnp.float32),
        mesh=_sc_mesh(),
        scratch_types=scratch,
        compiler_params=pltpu.CompilerParams(use_tc_tiling_on_sc=False),
    )
    def deg_kernel(dst_hbm, deg_hbm, ones_v, zeros_v, idx_v, acc):
        c = lax.axis_index("c")
        s = lax.axis_index("s")
        _fill_const(ones_v, 128, 16, 1.0)
        _fill_const(zeros_v, 128, 16, 0.0)

        def zslab(j, carry):
            pltpu.sync_copy(zeros_v, acc.at[pl.ds(s * ROWS_PT + j * 128, 128)])
            return carry
        lax.fori_loop(0, ROWS_PT // 128, zslab, 0)
        plsc.subcore_barrier()

        pltpu.sync_copy(dst_hbm.at[pl.ds(c * NBT + s * BPT, BPT)], idx_v)

        def scat(i, carry):
            pltpu.sync_copy(ones_v, acc.at[idx_v.at[i]], add=True)
            return carry
        lax.fori_loop(0, BPT, scat, 0)
        plsc.subcore_barrier()

        pltpu.sync_copy(
            acc.at[pl.ds(s * ROWS_PT, ROWS_PT)],
            deg_hbm.at[pl.ds(c * N_PAD + s * ROWS_PT, ROWS_PT)])

    return deg_kernel(dst_all)


def _make_agg(nch):
    nvi = 2 * nch
    NB = 5
    scratch = (
        [
            pltpu.VMEM((BPT, 128), jnp.int32),
            pltpu.VMEM((BPT, 128), jnp.int32),
        ]
        + [pltpu.VMEM((128, FW), jnp.float32) for _ in range(NB)]
        + [pltpu.MemorySpace.VMEM_SHARED((N_PAD, FW), jnp.float32),
           pltpu.MemorySpace.VMEM_SHARED((N_PAD, FW), jnp.float32)]
        + [pltpu.SemaphoreType.DMA for _ in range(2 * NB)]
    )

    @functools.partial(
        pl.kernel,
        out_type=jax.ShapeDtypeStruct((nvi, N_PAD, FW), jnp.float32),
        mesh=_sc_mesh(),
        scratch_types=scratch,
        compiler_params=pltpu.CompilerParams(use_tc_tiling_on_sc=False),
    )
    def agg_kernel(u_hbm, src_hbm, dst_hbm, out_hbm, *refs):
        srcb, dstb = refs[0], refs[1]
        rows = refs[2:2 + NB]
        acc = refs[2 + NB]
        usp = refs[3 + NB]
        gsem = refs[4 + NB:4 + 2 * NB]
        ssem = refs[4 + 2 * NB:4 + 3 * NB]
        c = lax.axis_index("c")
        s = lax.axis_index("s")
        base_b = c * NBT + s * BPT
        pltpu.sync_copy(src_hbm.at[pl.ds(base_b, BPT)], srcb)
        pltpu.sync_copy(dst_hbm.at[pl.ds(base_b, BPT)], dstb)

        def do_chunk(i, carry):
            vi = c * nch + i
            uview = u_hbm.at[vi]

            pltpu.sync_copy(uview.at[pl.ds(s * ROWS_PT, ROWS_PT)],
                            usp.at[pl.ds(s * ROWS_PT, ROWS_PT)])

            _fill_const(rows[NB - 1], 128, FW, 0.0)

            def zslab(j, c2):
                pltpu.sync_copy(rows[NB - 1],
                                acc.at[pl.ds(s * ROWS_PT + j * 128, 128)])
                return c2
            lax.fori_loop(0, ROWS_PT // 128, zslab, 0)
            plsc.subcore_barrier()

            for t in range(4):
                pltpu.async_copy(usp.at[srcb.at[t]], rows[t], gsem[t])

            def grp(j, c2):
                b0 = NB * j
                for u in range(NB):
                    b = b0 + u
                    tn = (u + 4) % NB

                    pltpu.make_async_copy(
                        usp.at[srcb.at[b]], rows[u], gsem[u]).wait()
                    pltpu.async_copy(rows[u], acc.at[dstb.at[b]], ssem[u],
                                     add=True)

                    @pl.when(b < BPT - 4)
                    def _prefetch():
                        @pl.when(b >= 1)
                        def _drain_prev():
                            pltpu.make_async_copy(
                                rows[tn], acc.at[dstb.at[b - 1]],
                                ssem[tn]).wait()
                        pltpu.async_copy(
                            usp.at[srcb.at[b + 4]], rows[tn], gsem[tn])
                return c2
            lax.fori_loop(0, BPT // NB, grp, 0)
            for t in range(NB):
                pltpu.make_async_copy(
                    rows[t], acc.at[dstb.at[BPT - NB + t]], ssem[t]).wait()
            plsc.subcore_barrier()

            pltpu.sync_copy(acc.at[pl.ds(s * ROWS_PT, ROWS_PT)],
                            out_hbm.at[vi].at[pl.ds(s * ROWS_PT, ROWS_PT)])
            plsc.subcore_barrier()
            return carry

        lax.fori_loop(0, nch, do_chunk, 0)

    return agg_kernel


def _embed_call(x_pad, w, b2d, deg1, deg2):
    def body(x_ref, w_ref, b_ref, d1_ref, d2_ref,
             h_ref, v_ref, i1_ref, i2_ref):
        i = pl.program_id(0)
        h = jnp.dot(x_ref[...], w_ref[...], preferred_element_type=jnp.float32)
        h = jnp.maximum(h + b_ref[...], 0.0)
        row = i * RB + lax.broadcasted_iota(jnp.int32, (RB, 16), 0)
        valid = row < NN
        d1 = d1_ref[...]
        d2 = d2_ref[...]
        dinv1 = jnp.where((d1 > 0) & valid, lax.rsqrt(d1), 0.0)
        dinv2 = jnp.where((d2 > 0) & valid, lax.rsqrt(d2), 0.0)
        h_ref[...] = h
        i1_ref[...] = dinv1
        i2_ref[...] = dinv2
        c1 = dinv1[:, :1]
        c2 = dinv2[:, :1]
        for k in range(4):
            v_ref[0, k] = h[:, k * FW:(k + 1) * FW] * c1
            v_ref[1, k] = h[:, k * FW:(k + 1) * FW] * c2

    return pl.pallas_call(
        body,
        grid=(GRID_R,),
        in_specs=[
            pl.BlockSpec((RB, 128), lambda i: (i, 0)),
            pl.BlockSpec((128, 128), lambda i: (0, 0)),
            pl.BlockSpec((1, 128), lambda i: (0, 0)),
            pl.BlockSpec((RB, 16), lambda i: (i, 0)),
            pl.BlockSpec((RB, 16), lambda i: (i, 0)),
        ],
        out_specs=[
            pl.BlockSpec((RB, 128), lambda i: (i, 0)),
            pl.BlockSpec((2, 4, RB, FW), lambda i: (0, 0, i, 0)),
            pl.BlockSpec((RB, 16), lambda i: (i, 0)),
            pl.BlockSpec((RB, 16), lambda i: (i, 0)),
        ],
        out_shape=[
            jax.ShapeDtypeStruct((N_PAD, 128), jnp.float32),
            jax.ShapeDtypeStruct((2, 4, N_PAD, FW), jnp.float32),
            jax.ShapeDtypeStruct((N_PAD, 16), jnp.float32),
            jax.ShapeDtypeStruct((N_PAD, 16), jnp.float32),
        ],
    )(x_pad, w, b2d, deg1, deg2)


def _scale_bn_call(nvi, acc, dinv1, dinv2, gamma3d, beta3d):
    bn_scale = float(1.0 / np.sqrt(1.0 + BN_EPS))

    def body(a_ref, d1_ref, d2_ref, g_ref, b_ref, z_ref, v_ref):
        ch = pl.program_id(0)
        a = jnp.concatenate([a_ref[k] for k in range(4)], axis=1)
        d1 = d1_ref[...][:, :1]
        d2 = d2_ref[...][:, :1]
        dsel = jnp.where(2 * ch < nvi, d1, d2)
        z = a * dsel * (g_ref[0, 0] * bn_scale) + b_ref[0, 0]
        z_ref[0] = z
        for k in range(4):
            v_ref[0, k] = z[:, k * FW:(k + 1) * FW] * d1
            v_ref[1, k] = z[:, k * FW:(k + 1) * FW] * d2

    return pl.pallas_call(
        body,
        grid=(nvi, GRID_R),
        in_specs=[
            pl.BlockSpec((4, RB, FW), lambda ch, r: (ch, r, 0)),
            pl.BlockSpec((RB, 16), lambda ch, r: (r, 0)),
            pl.BlockSpec((RB, 16), lambda ch, r: (r, 0)),
            pl.BlockSpec((1, 1, 128), lambda ch, r: (ch, 0, 0)),
            pl.BlockSpec((1, 1, 128), lambda ch, r: (ch, 0, 0)),
        ],
        out_specs=[
            pl.BlockSpec((1, RB, 128), lambda ch, r: (ch, r, 0)),
            pl.BlockSpec((2, 4, RB, FW), lambda ch, r: (0, ch, r, 0)),
        ],
        out_shape=[
            jax.ShapeDtypeStruct((nvi, N_PAD, 128), jnp.float32),
            jax.ShapeDtypeStruct((2, 4 * nvi, N_PAD, FW), jnp.float32),
        ],
    )(acc, dinv1, dinv2, gamma3d, beta3d)


def _bn2_project(acc, dinv1, dinv2, g32, b32, wa):
    bn_scale = float(1.0 / np.sqrt(1.0 + BN_EPS))

    def body(a_ref, d1_ref, d2_ref, g_ref, b_ref, wa_ref, z_ref, v_ref):
        d1 = d1_ref[...][:, :1]
        d2 = d2_ref[...][:, :1]
        p1 = jnp.zeros((RB, 128), jnp.float32)
        p2 = jnp.zeros((RB, 128), jnp.float32)
        zs = []
        for k in range(16):
            dsel = d1 if k < 8 else d2
            z32 = a_ref[k] * dsel * (g_ref[k] * bn_scale) + b_ref[k]
            zs.append(z32)
            p1 += jnp.dot(z32, wa_ref[k], preferred_element_type=jnp.float32)
            p2 += jnp.dot(z32, wa_ref[16 + k],
                          preferred_element_type=jnp.float32)
        for ch in range(4):
            z_ref[ch] = jnp.concatenate(zs[4 * ch:4 * ch + 4], axis=1)
        for k in range(4):
            v_ref[0, k] = p1[:, k * FW:(k + 1) * FW] * d1
            v_ref[1, k] = p2[:, k * FW:(k + 1) * FW] * d2

    return pl.pallas_call(
        body,
        grid=(GRID_R,),
        in_specs=[
            pl.BlockSpec((16, RB, FW), lambda i: (0, i, 0)),
            pl.BlockSpec((RB, 16), lambda i: (i, 0)),
            pl.BlockSpec((RB, 16), lambda i: (i, 0)),
            pl.BlockSpec((16, 1, FW), lambda i: (0, 0, 0)),
            pl.BlockSpec((16, 1, FW), lambda i: (0, 0, 0)),
            pl.BlockSpec((32, FW, 128), lambda i: (0, 0, 0)),
        ],
        out_specs=[
            pl.BlockSpec((4, RB, 128), lambda i: (0, i, 0)),
            pl.BlockSpec((2, 4, RB, FW), lambda i: (0, 0, i, 0)),
        ],
        out_shape=[
            jax.ShapeDtypeStruct((4, N_PAD, 128), jnp.float32),
            jax.ShapeDtypeStruct((2, 4, N_PAD, FW), jnp.float32),
        ],
    )(acc, dinv1, dinv2, g32, b32, wa)


def _final_call(h, z1, z2, a3, dinv1, dinv2, w7, b2d):
    def body(h_ref, z1_ref, z2_ref, a3_ref, d1_ref, d2_ref, w7_ref,
             b_ref, o_ref):
        s = jnp.dot(h_ref[...], w7_ref[0], preferred_element_type=jnp.float32)
        for k in range(2):
            s += jnp.dot(z1_ref[k], w7_ref[1 + k],
                         preferred_element_type=jnp.float32)
        for k in range(4):
            s += jnp.dot(z2_ref[k], w7_ref[3 + k],
                         preferred_element_type=jnp.float32)
        d1 = d1_ref[...][:, :1]
        d2 = d2_ref[...][:, :1]
        q1 = jnp.concatenate([a3_ref[k] for k in range(4)], axis=1)
        q2 = jnp.concatenate([a3_ref[4 + k] for k in range(4)], axis=1)
        o_ref[...] = s + q1 * d1 + q2 * d2 + b_ref[...]

    return pl.pallas_call(
        body,
        grid=(GRID_R,),
        in_specs=[
            pl.BlockSpec((RB, 128), lambda i: (i, 0)),
            pl.BlockSpec((2, RB, 128), lambda i: (0, i, 0)),
            pl.BlockSpec((4, RB, 128), lambda i: (0, i, 0)),
            pl.BlockSpec((8, RB, FW), lambda i: (0, i, 0)),
            pl.BlockSpec((RB, 16), lambda i: (i, 0)),
            pl.BlockSpec((RB, 16), lambda i: (i, 0)),
            pl.BlockSpec((7, 128, 128), lambda i: (0, 0, 0)),
            pl.BlockSpec((1, 128), lambda i: (0, 0)),
        ],
        out_specs=pl.BlockSpec((RB, 128), lambda i: (i, 0)),
        out_shape=jax.ShapeDtypeStruct((N_PAD, 128), jnp.float32),
    )(h, z1, z2, a3, dinv1, dinv2, w7, b2d)


def _prep_edges(edge_index):
    pad = E_PAD - EE
    src = jnp.concatenate(
        [edge_index[0], jnp.full((pad,), NN, jnp.int32)]).reshape(NBT, 128)
    dst = jnp.concatenate(
        [edge_index[1], jnp.full((pad,), NN, jnp.int32)]).reshape(NBT, 128)
    return src, dst


def kernel(x, edge_index, edge_index2, W_embed, b_embed,
           bn0_gamma, bn0_beta, bn1_gamma, bn1_beta, W_out, b_out):
    s1, d1 = _prep_edges(edge_index)
    s2, d2 = _prep_edges(edge_index2)
    src_all = jnp.concatenate([s1, s2], axis=0)
    dst_all = jnp.concatenate([d1, d2], axis=0)

    deg_flat = _deg_call(dst_all)
    deg1 = deg_flat[:N_PAD]
    deg2 = deg_flat[N_PAD:]

    x_pad = jnp.pad(x, ((0, N_PAD - NN), (0, 0)))
    h, v0, dinv1, dinv2 = _embed_call(
        x_pad, W_embed, b_embed.reshape(1, 128), deg1, deg2)

    agg4 = _make_agg(4)
    acc1 = agg4(v0.reshape(8, N_PAD, FW), src_all, dst_all)
    z1, v1 = _scale_bn_call(2, acc1, dinv1, dinv2,
                            bn0_gamma.reshape(2, 1, 128),
                            bn0_beta.reshape(2, 1, 128))

    acc2 = _make_agg(8)(v1.reshape(16, N_PAD, FW), src_all, dst_all)
    z2, v3 = _bn2_project(acc2, dinv1, dinv2,
                          bn1_gamma.reshape(16, 1, FW),
                          bn1_beta.reshape(16, 1, FW),
                          W_out[896:].reshape(32, FW, 128))

    acc3 = agg4(v3.reshape(8, N_PAD, FW), src_all, dst_all)

    out = _final_call(h, z1, z2, acc3, dinv1, dinv2,
                      W_out[:896].reshape(7, 128, 128),
                      b_out.reshape(1, 128))
    return out[:NN]


# speedup vs baseline: 1.6829x; 1.0030x over previous
import functools

import jax
import jax.numpy as jnp
import numpy as np
from jax import lax
from jax.experimental import pallas as pl
from jax.experimental.pallas import tpu as pltpu
from jax.experimental.pallas import tpu_sc as plsc

NN = 10000
EE = 320000
N_PAD = 10240
ROWS_PT = N_PAD // 16
BPT = 160
NBT = 16 * BPT
E_PAD = NBT * 128
FW = 32
RB = 512
GRID_R = N_PAD // RB
BN_EPS = 1e-5


def _sc_mesh():
    return plsc.VectorSubcoreMesh(core_axis_name="c", subcore_axis_name="s")


def _fill_const(buf, rows, cols, val):
    def frow(r, carry):
        def fcol(k, c2):
            buf[r, pl.ds(k * 16, 16)] = jnp.full((16,), val, jnp.float32)
            return c2
        return lax.fori_loop(0, cols // 16, fcol, carry)
    lax.fori_loop(0, rows, frow, 0)


def _deg_call(dst_all):
    scratch = [
        pltpu.VMEM((128, 16), jnp.float32),
        pltpu.VMEM((128, 16), jnp.float32),
        pltpu.VMEM((BPT, 128), jnp.int32),
        pltpu.MemorySpace.VMEM_SHARED((N_PAD, 16), jnp.float32),
    ]

    @functools.partial(
        pl.kernel,
        out_type=jax.ShapeDtypeStruct((2 * N_PAD, 16), jnp.float32),
        mesh=_sc_mesh(),
        scratch_types=scratch,
        compiler_params=pltpu.CompilerParams(use_tc_tiling_on_sc=False),
    )
    def deg_kernel(dst_hbm, deg_hbm, ones_v, zeros_v, idx_v, acc):
        c = lax.axis_index("c")
        s = lax.axis_index("s")
        _fill_const(ones_v, 128, 16, 1.0)
        _fill_const(zeros_v, 128, 16, 0.0)

        def zslab(j, carry):
            pltpu.sync_copy(zeros_v, acc.at[pl.ds(s * ROWS_PT + j * 128, 128)])
            return carry
        lax.fori_loop(0, ROWS_PT // 128, zslab, 0)
        plsc.subcore_barrier()

        pltpu.sync_copy(dst_hbm.at[pl.ds(c * NBT + s * BPT, BPT)], idx_v)

        def scat(i, carry):
            pltpu.sync_copy(ones_v, acc.at[idx_v.at[i]], add=True)
            return carry
        lax.fori_loop(0, BPT, scat, 0)
        plsc.subcore_barrier()

        pltpu.sync_copy(
            acc.at[pl.ds(s * ROWS_PT, ROWS_PT)],
            deg_hbm.at[pl.ds(c * N_PAD + s * ROWS_PT, ROWS_PT)])

    return deg_kernel(dst_all)


def _make_agg(nch):
    nvi = 2 * nch
    NB = 5
    scratch = (
        [
            pltpu.VMEM((BPT, 128), jnp.int32),
            pltpu.VMEM((BPT, 128), jnp.int32),
        ]
        + [pltpu.VMEM((128, FW), jnp.float32) for _ in range(NB)]
        + [pltpu.MemorySpace.VMEM_SHARED((N_PAD, FW), jnp.float32),
           pltpu.MemorySpace.VMEM_SHARED((N_PAD, FW), jnp.float32)]
        + [pltpu.SemaphoreType.DMA for _ in range(2 * NB)]
    )

    @functools.partial(
        pl.kernel,
        out_type=jax.ShapeDtypeStruct((nvi, N_PAD, FW), jnp.float32),
        mesh=_sc_mesh(),
        scratch_types=scratch,
        compiler_params=pltpu.CompilerParams(use_tc_tiling_on_sc=False),
    )
    def agg_kernel(u_hbm, src_hbm, dst_hbm, out_hbm, *refs):
        srcb, dstb = refs[0], refs[1]
        rows = refs[2:2 + NB]
        acc = refs[2 + NB]
        usp = refs[3 + NB]
        gsem = refs[4 + NB:4 + 2 * NB]
        ssem = refs[4 + 2 * NB:4 + 3 * NB]
        c = lax.axis_index("c")
        s = lax.axis_index("s")
        base_b = c * NBT + s * BPT
        pltpu.sync_copy(src_hbm.at[pl.ds(base_b, BPT)], srcb)
        pltpu.sync_copy(dst_hbm.at[pl.ds(base_b, BPT)], dstb)

        def do_chunk(i, carry):
            vi = c * nch + i
            uview = u_hbm.at[vi]

            pltpu.sync_copy(uview.at[pl.ds(s * ROWS_PT, ROWS_PT)],
                            usp.at[pl.ds(s * ROWS_PT, ROWS_PT)])

            _fill_const(rows[NB - 1], 128, FW, 0.0)

            def zslab(j, c2):
                pltpu.sync_copy(rows[NB - 1],
                                acc.at[pl.ds(s * ROWS_PT + j * 128, 128)])
                return c2
            lax.fori_loop(0, ROWS_PT // 128, zslab, 0)
            plsc.subcore_barrier()

            for t in range(4):
                pltpu.async_copy(usp.at[srcb.at[t]], rows[t], gsem[t])

            def grp(j, c2):
                b0 = NB * j
                for u in range(NB):
                    b = b0 + u
                    tn = (u + 4) % NB

                    pltpu.make_async_copy(
                        usp.at[srcb.at[b]], rows[u], gsem[u]).wait()
                    pltpu.async_copy(rows[u], acc.at[dstb.at[b]], ssem[u],
                                     add=True)

                    @pl.when(b < BPT - 4)
                    def _prefetch():
                        @pl.when(b >= 1)
                        def _drain_prev():
                            pltpu.make_async_copy(
                                rows[tn], acc.at[dstb.at[b - 1]],
                                ssem[tn]).wait()
                        pltpu.async_copy(
                            usp.at[srcb.at[b + 4]], rows[tn], gsem[tn])
                return c2
            lax.fori_loop(0, BPT // NB, grp, 0)
            for t in range(NB):
                pltpu.make_async_copy(
                    rows[t], acc.at[dstb.at[BPT - NB + t]], ssem[t]).wait()
            plsc.subcore_barrier()

            pltpu.sync_copy(acc.at[pl.ds(s * ROWS_PT, ROWS_PT)],
                            out_hbm.at[vi].at[pl.ds(s * ROWS_PT, ROWS_PT)])
            plsc.subcore_barrier()
            return carry

        lax.fori_loop(0, nch, do_chunk, 0)

    return agg_kernel


def _embed_call(x_pad, w, b2d, deg1, deg2):
    def body(x_ref, w_ref, b_ref, d1_ref, d2_ref,
             h_ref, v_ref, i1_ref, i2_ref):
        i = pl.program_id(0)
        h = jnp.dot(x_ref[...], w_ref[...], preferred_element_type=jnp.float32)
        h = jnp.maximum(h + b_ref[...], 0.0)
        row = i * RB + lax.broadcasted_iota(jnp.int32, (RB, 16), 0)
        valid = row < NN
        d1 = d1_ref[...]
        d2 = d2_ref[...]
        dinv1 = jnp.where((d1 > 0) & valid, lax.rsqrt(d1), 0.0)
        dinv2 = jnp.where((d2 > 0) & valid, lax.rsqrt(d2), 0.0)
        h_ref[...] = h
        i1_ref[...] = dinv1
        i2_ref[...] = dinv2
        c1 = dinv1[:, :1]
        c2 = dinv2[:, :1]
        for k in range(4):
            v_ref[0, k] = h[:, k * FW:(k + 1) * FW] * c1
            v_ref[1, k] = h[:, k * FW:(k + 1) * FW] * c2

    return pl.pallas_call(
        body,
        grid=(GRID_R,),
        in_specs=[
            pl.BlockSpec((RB, 128), lambda i: (i, 0)),
            pl.BlockSpec((128, 128), lambda i: (0, 0)),
            pl.BlockSpec((1, 128), lambda i: (0, 0)),
            pl.BlockSpec((RB, 16), lambda i: (i, 0)),
            pl.BlockSpec((RB, 16), lambda i: (i, 0)),
        ],
        out_specs=[
            pl.BlockSpec((RB, 128), lambda i: (i, 0)),
            pl.BlockSpec((2, 4, RB, FW), lambda i: (0, 0, i, 0)),
            pl.BlockSpec((RB, 16), lambda i: (i, 0)),
            pl.BlockSpec((RB, 16), lambda i: (i, 0)),
        ],
        out_shape=[
            jax.ShapeDtypeStruct((N_PAD, 128), jnp.float32),
            jax.ShapeDtypeStruct((2, 4, N_PAD, FW), jnp.float32),
            jax.ShapeDtypeStruct((N_PAD, 16), jnp.float32),
            jax.ShapeDtypeStruct((N_PAD, 16), jnp.float32),
        ],
    )(x_pad, w, b2d, deg1, deg2)


def _prescale_bn_call(nvi, acc, dinv1, dinv2, gamma3d, beta3d):
    bn_scale = float(1.0 / np.sqrt(1.0 + BN_EPS))

    def body(a_ref, d1_ref, d2_ref, g_ref, b_ref, v_ref):
        ch = pl.program_id(0)
        a = jnp.concatenate([a_ref[k] for k in range(4)], axis=1)
        d1 = d1_ref[...][:, :1]
        d2 = d2_ref[...][:, :1]
        dsel = jnp.where(2 * ch < nvi, d1, d2)
        z = a * dsel * (g_ref[0, 0] * bn_scale) + b_ref[0, 0]
        for k in range(4):
            v_ref[0, k] = z[:, k * FW:(k + 1) * FW] * d1
            v_ref[1, k] = z[:, k * FW:(k + 1) * FW] * d2

    return pl.pallas_call(
        body,
        grid=(nvi, GRID_R),
        in_specs=[
            pl.BlockSpec((4, RB, FW), lambda ch, r: (ch, r, 0)),
            pl.BlockSpec((RB, 16), lambda ch, r: (r, 0)),
            pl.BlockSpec((RB, 16), lambda ch, r: (r, 0)),
            pl.BlockSpec((1, 1, 128), lambda ch, r: (ch, 0, 0)),
            pl.BlockSpec((1, 1, 128), lambda ch, r: (ch, 0, 0)),
        ],
        out_specs=pl.BlockSpec((2, 4, RB, FW), lambda ch, r: (0, ch, r, 0)),
        out_shape=jax.ShapeDtypeStruct((2, 4 * nvi, N_PAD, FW), jnp.float32),
    )(acc, dinv1, dinv2, gamma3d, beta3d)


def _z_bn_call(nvi, acc, dinv1, dinv2, gamma3d, beta3d):
    bn_scale = float(1.0 / np.sqrt(1.0 + BN_EPS))

    def body(a_ref, d1_ref, d2_ref, g_ref, b_ref, z_ref):
        ch = pl.program_id(0)
        a = jnp.concatenate([a_ref[k] for k in range(4)], axis=1)
        d1 = d1_ref[...][:, :1]
        d2 = d2_ref[...][:, :1]
        dsel = jnp.where(2 * ch < nvi, d1, d2)
        z_ref[0] = a * dsel * (g_ref[0, 0] * bn_scale) + b_ref[0, 0]

    return pl.pallas_call(
        body,
        grid=(nvi, GRID_R),
        in_specs=[
            pl.BlockSpec((4, RB, FW), lambda ch, r: (ch, r, 0)),
            pl.BlockSpec((RB, 16), lambda ch, r: (r, 0)),
            pl.BlockSpec((RB, 16), lambda ch, r: (r, 0)),
            pl.BlockSpec((1, 1, 128), lambda ch, r: (ch, 0, 0)),
            pl.BlockSpec((1, 1, 128), lambda ch, r: (ch, 0, 0)),
        ],
        out_specs=pl.BlockSpec((1, RB, 128), lambda ch, r: (ch, r, 0)),
        out_shape=jax.ShapeDtypeStruct((nvi, N_PAD, 128), jnp.float32),
    )(acc, dinv1, dinv2, gamma3d, beta3d)


def _bn2_project(acc, dinv1, dinv2, g32, b32, wa):
    bn_scale = float(1.0 / np.sqrt(1.0 + BN_EPS))

    def body(a_ref, d1_ref, d2_ref, g_ref, b_ref, wa_ref, v_ref):
        d1 = d1_ref[...][:, :1]
        d2 = d2_ref[...][:, :1]
        p1 = jnp.zeros((RB, 128), jnp.float32)
        p2 = jnp.zeros((RB, 128), jnp.float32)
        for k in range(16):
            dsel = d1 if k < 8 else d2
            z32 = a_ref[k] * dsel * (g_ref[k] * bn_scale) + b_ref[k]
            p1 += jnp.dot(z32, wa_ref[k], preferred_element_type=jnp.float32)
            p2 += jnp.dot(z32, wa_ref[16 + k],
                          preferred_element_type=jnp.float32)
        for k in range(4):
            v_ref[0, k] = p1[:, k * FW:(k + 1) * FW] * d1
            v_ref[1, k] = p2[:, k * FW:(k + 1) * FW] * d2

    return pl.pallas_call(
        body,
        grid=(GRID_R,),
        in_specs=[
            pl.BlockSpec((16, RB, FW), lambda i: (0, i, 0)),
            pl.BlockSpec((RB, 16), lambda i: (i, 0)),
            pl.BlockSpec((RB, 16), lambda i: (i, 0)),
            pl.BlockSpec((16, 1, FW), lambda i: (0, 0, 0)),
            pl.BlockSpec((16, 1, FW), lambda i: (0, 0, 0)),
            pl.BlockSpec((32, FW, 128), lambda i: (0, 0, 0)),
        ],
        out_specs=pl.BlockSpec((2, 4, RB, FW), lambda i: (0, 0, i, 0)),
        out_shape=jax.ShapeDtypeStruct((2, 4, N_PAD, FW), jnp.float32),
    )(acc, dinv1, dinv2, g32, b32, wa)


def _final_call(h, z1, z2, a3, dinv1, dinv2, w7, b2d):
    def body(h_ref, z1_ref, z2_ref, a3_ref, d1_ref, d2_ref, w7_ref,
             b_ref, o_ref):
        s = jnp.dot(h_ref[...], w7_ref[0], preferred_element_type=jnp.float32)
        for k in range(2):
            s += jnp.dot(z1_ref[k], w7_ref[1 + k],
                         preferred_element_type=jnp.float32)
        for k in range(4):
            s += jnp.dot(z2_ref[k], w7_ref[3 + k],
                         preferred_element_type=jnp.float32)
        d1 = d1_ref[...][:, :1]
        d2 = d2_ref[...][:, :1]
        q1 = jnp.concatenate([a3_ref[k] for k in range(4)], axis=1)
        q2 = jnp.concatenate([a3_ref[4 + k] for k in range(4)], axis=1)
        o_ref[...] = s + q1 * d1 + q2 * d2 + b_ref[...]

    return pl.pallas_call(
        body,
        grid=(GRID_R,),
        in_specs=[
            pl.BlockSpec((RB, 128), lambda i: (i, 0)),
            pl.BlockSpec((2, RB, 128), lambda i: (0, i, 0)),
            pl.BlockSpec((4, RB, 128), lambda i: (0, i, 0)),
            pl.BlockSpec((8, RB, FW), lambda i: (0, i, 0)),
            pl.BlockSpec((RB, 16), lambda i: (i, 0)),
            pl.BlockSpec((RB, 16), lambda i: (i, 0)),
            pl.BlockSpec((7, 128, 128), lambda i: (0, 0, 0)),
            pl.BlockSpec((1, 128), lambda i: (0, 0)),
        ],
        out_specs=pl.BlockSpec((RB, 128), lambda i: (i, 0)),
        out_shape=jax.ShapeDtypeStruct((N_PAD, 128), jnp.float32),
    )(h, z1, z2, a3, dinv1, dinv2, w7, b2d)


def _prep_edges(edge_index):
    pad = E_PAD - EE
    src = jnp.concatenate(
        [edge_index[0], jnp.full((pad,), NN, jnp.int32)]).reshape(NBT, 128)
    dst = jnp.concatenate(
        [edge_index[1], jnp.full((pad,), NN, jnp.int32)]).reshape(NBT, 128)
    return src, dst


def kernel(x, edge_index, edge_index2, W_embed, b_embed,
           bn0_gamma, bn0_beta, bn1_gamma, bn1_beta, W_out, b_out):
    s1, d1 = _prep_edges(edge_index)
    s2, d2 = _prep_edges(edge_index2)
    src_all = jnp.concatenate([s1, s2], axis=0)
    dst_all = jnp.concatenate([d1, d2], axis=0)

    deg_flat = _deg_call(dst_all)
    deg1 = deg_flat[:N_PAD]
    deg2 = deg_flat[N_PAD:]

    x_pad = jnp.pad(x, ((0, N_PAD - NN), (0, 0)))
    h, v0, dinv1, dinv2 = _embed_call(
        x_pad, W_embed, b_embed.reshape(1, 128), deg1, deg2)

    agg4 = _make_agg(4)
    acc1 = agg4(v0.reshape(8, N_PAD, FW), src_all, dst_all)
    v1 = _prescale_bn_call(2, acc1, dinv1, dinv2,
                           bn0_gamma.reshape(2, 1, 128),
                           bn0_beta.reshape(2, 1, 128))
    z1 = _z_bn_call(2, acc1, dinv1, dinv2,
                    bn0_gamma.reshape(2, 1, 128),
                    bn0_beta.reshape(2, 1, 128))

    acc2 = _make_agg(8)(v1.reshape(16, N_PAD, FW), src_all, dst_all)
    v3 = _bn2_project(acc2, dinv1, dinv2,
                      bn1_gamma.reshape(16, 1, FW),
                      bn1_beta.reshape(16, 1, FW),
                      W_out[896:].reshape(32, FW, 128))
    z2 = _z_bn_call(4, acc2, dinv1, dinv2,
                    bn1_gamma.reshape(4, 1, 128),
                    bn1_beta.reshape(4, 1, 128))

    acc3 = agg4(v3.reshape(8, N_PAD, FW), src_all, dst_all)

    out = _final_call(h, z1, z2, acc3, dinv1, dinv2,
                      W_out[:896].reshape(7, 128, 128),
                      b_out.reshape(1, 128))
    return out[:NN]
